# baseline (speedup 1.0000x reference)
import functools
import math

import jax
import jax.numpy as jnp
from jax import lax
from jax.experimental import pallas as pl
from jax.experimental.pallas import tpu as pltpu
from jax.experimental.pallas import tpu_sc as plsc

N = 10000
E = 160000
D = 128
C = 128
EDGE_DIM = 16
RADIUS = 5.0
EPS = 1e-5

BE = 640
BN_ = 400

NC = 2
NS = 16
NW = NC * NS

_f32 = jnp.float32


def _silu(x):
    return x * jax.nn.sigmoid(x)


_GCH = 128
_GW = 5120
_GNC = _GW // _GCH
_GLAST = (E - 31 * _GW) // _GCH
_GCR = E // _GCH + (_GNC - _GLAST)
_GNBUF = 4


def _sc_gather_one(table, idx_d3, idx_s3):
    dt = table.dtype
    W = table.shape[1]
    mesh = plsc.VectorSubcoreMesh(core_axis_name="c", subcore_axis_name="s")

    @functools.partial(
        pl.kernel,
        mesh=mesh,
        out_type=[jax.ShapeDtypeStruct((E // _GCH, _GCH, W), dt)
                  for _ in range(2)],
        scratch_types=[
            pltpu.VMEM((_GNC // 8, 8, _GCH), jnp.int32),
        ] + [pltpu.VMEM((_GCH, W), dt) for _ in range(_GNBUF)] + [
            pltpu.SemaphoreType.DMA for _ in range(2 * _GNBUF)
        ],
        compiler_params=pltpu.CompilerParams(use_tc_tiling_on_sc=False),
    )
    def gather_k(tab_h, id_h, is_h, o_d, o_s, idxb, *bufs_sems):
        rows = bufs_sems[:_GNBUF]
        gsem = bufs_sems[_GNBUF:2 * _GNBUF]
        wsem = bufs_sems[2 * _GNBUF:3 * _GNBUF]
        wid = lax.axis_index("s") * NC + lax.axis_index("c")
        full = wid < NW - 1
        crb = wid * (_GNC // 8)

        for idx_h, out_h in ((id_h, o_d), (is_h, o_s)):
            pltpu.sync_copy(idx_h.at[pl.ds(crb, _GNC // 8)], idxb)

            def wait_g(sl, out_h=out_h):
                pltpu.make_async_copy(tab_h.at[pl.ds(0, _GCH)],
                                      rows[sl], gsem[sl]).wait()

            def wait_w(sl, out_h=out_h):
                pltpu.make_async_copy(rows[sl], out_h.at[0], wsem[sl]).wait()

            def issue(ci, sl, out_h=out_h):
                pltpu.async_copy(tab_h.at[idxb.at[ci // 8, ci % 8]],
                                 rows[sl], gsem[sl])

            def drain(ci, sl, out_h=out_h):
                wait_g(sl)
                pltpu.async_copy(rows[sl], out_h.at[wid * _GNC + ci],
                                 wsem[sl])

            def guarded(ci, f):
                if ci < _GLAST:
                    f()
                else:
                    pl.when(full)(f)

            for ci in range(_GNC + _GNBUF):
                sl = ci % _GNBUF
                di = ci - _GNBUF
                if di >= 0:
                    guarded(di, lambda di=di, sl=sl: drain(di, sl))
                if ci < _GNC:
                    def start(ci=ci, sl=sl):
                        if ci >= _GNBUF:
                            wait_w(sl)
                        issue(ci, sl)
                    guarded(ci, start)
            for sl in range(_GNBUF):
                wait_w(sl)

    o_d, o_s = gather_k(table, idx_d3, idx_s3)
    return o_d, o_s


_SCH = 128
_ZR = 16
_NPT = 624
_NREM = N - _NPT * NS


def _sc_scatter_one(msg, dst2):
    mesh = plsc.VectorSubcoreMesh(core_axis_name="c", subcore_axis_name="s")

    @functools.partial(
        pl.kernel,
        mesh=mesh,
        out_type=[jax.ShapeDtypeStruct((2, N, D), _f32)],
        scratch_types=[
            pltpu.VMEM((_GNC // 8, 8, _SCH), jnp.int32),
            pltpu.VMEM((_SCH, D), _f32),
            pltpu.VMEM((_SCH, D), _f32),
            pltpu.VMEM((_ZR, D), _f32),
            pltpu.VMEM_SHARED((N, D), _f32),
            pltpu.SemaphoreType.DMA,
            pltpu.SemaphoreType.DMA,
        ],
    )
    def scatter_k(m_h, d_h, out_h, idxb, m0, m1, z_v, acc, l0, l1):
        cid = lax.axis_index("c")
        sid = lax.axis_index("s")
        wid = sid * NC + cid
        full = wid < NW - 1
        for r in range(_ZR):
            for cc in range(D // 16):
                z_v[r, pl.ds(cc * 16, 16)] = jnp.zeros((16,), _f32)
        row0 = pl.multiple_of(sid * _NPT, 8)

        def zbody(j, carry):
            pltpu.sync_copy(z_v, acc.at[pl.ds(row0 + j * _ZR, _ZR)])
            return carry
        lax.fori_loop(0, _NPT // _ZR, zbody, 0)

        @pl.when(sid == NS - 1)
        def _():
            pltpu.sync_copy(z_v, acc.at[pl.ds(_NPT * NS, _NREM)])
        plsc.subcore_barrier()

        crb = wid * (_GNC // 8)
        pltpu.sync_copy(d_h.at[pl.ds(crb, _GNC // 8)], idxb)
        mb = (m0, m1)
        ls = (l0, l1)

        def load(ci, sl):
            pltpu.async_copy(m_h.at[wid * _GNC + ci], mb[sl], ls[sl])

        def scat(ci, sl):
            pltpu.make_async_copy(m_h.at[0], mb[sl], ls[sl]).wait()
            pltpu.sync_copy(mb[sl], acc.at[idxb.at[ci // 8, ci % 8]],
                            add=True)

        def guarded(ci, f):
            if ci < _GLAST:
                f()
            else:
                pl.when(full)(f)

        for ci in range(_GNC + 1):
            sl = ci % 2
            if ci < _GNC:
                guarded(ci, lambda ci=ci, sl=sl: load(ci, sl))
            di = ci - 1
            if di >= 0:
                guarded(di, lambda di=di, sl=1 - sl: scat(di, sl))

        plsc.subcore_barrier()

        pltpu.sync_copy(acc.at[pl.ds(row0, _NPT)],
                        out_h.at[cid, pl.ds(row0, _NPT)])

        @pl.when(sid == NS - 1)
        def _():
            pltpu.sync_copy(acc.at[pl.ds(_NPT * NS, _NREM)],
                            out_h.at[cid, pl.ds(_NPT * NS, _NREM)])

    return scatter_k(msg, dst2)[0]


def _full(shape):
    return pl.BlockSpec(shape, lambda i: tuple(0 for _ in shape))


def _cart_edge1(xd, xs, e, w1d, w1s, w1e, b1c, w2g, b2g, w2a, b2a):
    grid = E // BE
    bf = jnp.bfloat16

    def body(xd_r, xs_r, e_r, w1d_r, w1s_r, w1e_r, b1_r, w2g_r, b2g_r,
             w2a_r, b2a_r, eg_r, mp_r, eo_r, st_r):
        e_ = e_r[...]
        h = (jnp.dot(xd_r[...], w1d_r[...], preferred_element_type=_f32)
             + jnp.dot(xs_r[...], w1s_r[...], preferred_element_type=_f32)
             + jnp.dot(e_.astype(bf), w1e_r[...],
                       preferred_element_type=_f32)
             + b1_r[...])
        eg = jnp.dot(_silu(h[:, :D]).astype(bf), w2g_r[...],
                     preferred_element_type=_f32) + b2g_r[...]
        mp = jnp.dot(_silu(h[:, D:]).astype(bf), w2a_r[...],
                     preferred_element_type=_f32) + b2a_r[...]
        eg_r[...] = eg.astype(bf)
        mp_r[...] = mp.astype(bf)
        eo_r[...] = e_ + eg

        @pl.when(pl.program_id(0) == 0)
        def _():
            st_r[...] = jnp.zeros((8, D), _f32)
        upd = jnp.concatenate(
            [jnp.sum(eg, axis=0)[None], jnp.sum(eg * eg, axis=0)[None],
             jnp.zeros((6, D), _f32)], axis=0)
        st_r[...] += upd

    return pl.pallas_call(
        body,
        grid=(grid,),
        in_specs=[
            pl.BlockSpec((BE, D), lambda i: (i, 0)),
            pl.BlockSpec((BE, D), lambda i: (i, 0)),
            pl.BlockSpec((BE, D), lambda i: (i, 0)),
            _full((D, 2 * D)), _full((D, 2 * D)), _full((D, 2 * D)),
            _full((1, 2 * D)),
            _full((D, D)), _full((1, D)), _full((D, D)), _full((1, D)),
        ],
        out_specs=[
            pl.BlockSpec((BE, D), lambda i: (i, 0)),
            pl.BlockSpec((BE, D), lambda i: (i, 0)),
            pl.BlockSpec((BE, D), lambda i: (i, 0)),
            pl.BlockSpec((8, D), lambda i: (0, 0)),
        ],
        out_shape=[
            jax.ShapeDtypeStruct((E, D), bf),
            jax.ShapeDtypeStruct((E, D), bf),
            jax.ShapeDtypeStruct((E, D), _f32),
            jax.ShapeDtypeStruct((8, D), _f32),
        ],
    )(xd, xs, e, w1d, w1s, w1e, b1c, w2g, b2g, w2a, b2a)


def _cart_edge2(e_gate, msg_pre, dist2, stats, bn_s, bn_b):
    grid = E // BE

    def body(eg_r, mp_r, d_r, st_r, s_r, b_r, msg_r):
        eg = eg_r[...].astype(_f32)
        mean = st_r[0:1, :] * (1.0 / E)
        var = st_r[1:2, :] * (1.0 / E) - mean * mean
        rstd = lax.rsqrt(var + EPS)
        xhat = (eg - mean) * rstd * s_r[...] + b_r[...]
        d = d_r[...]
        env = 0.5 * (jnp.cos((jnp.pi / RADIUS) * d) + 1.0)
        env = jnp.where(d <= RADIUS, env, 0.0)
        msg = mp_r[...].astype(_f32) * (jax.nn.sigmoid(xhat) * env)
        msg_r[...] = msg.reshape(BE // _GCH, _GCH, D)

    return pl.pallas_call(
        body,
        grid=(grid,),
        in_specs=[
            pl.BlockSpec((BE, D), lambda i: (i, 0)),
            pl.BlockSpec((BE, D), lambda i: (i, 0)),
            pl.BlockSpec((BE, 1), lambda i: (i, 0)),
            _full((8, D)), _full((1, D)), _full((1, D)),
        ],
        out_specs=[
            pl.BlockSpec((BE // _GCH, _GCH, D), lambda i: (i, 0, 0)),
        ],
        out_shape=[
            jax.ShapeDtypeStruct((E // _GCH, _GCH, D), _f32),
        ],
    )(e_gate, msg_pre, dist2, stats, bn_s, bn_b)[0]


def _mat_edge(xd, xs, ea, wqkv, bqkv, wkv, bkv, we, be,
              mu_d, mu_s, mu_e, mu_b, lna_s, lna_b, ml_w, ml_b,
              lnm_s, lnm_b):
    grid = E // BE
    scale = 1.0 / math.sqrt(3.0 * C)

    def body(xd_r, xs_r, ea_r, wqkv_r, bqkv_r, wkv_r, bkv_r,
             we_r, be_r, mud_r, mus_r, mue_r, mub_r, lnas_r, lnab_r,
             mlw_r, mlb_r, lnms_r, lnmb_r, out_r):
        bf = jnp.bfloat16
        xd_ = xd_r[...]
        xs_ = xs_r[...]
        qkv = jnp.dot(xd_, wqkv_r[...], preferred_element_type=_f32) + bqkv_r[...]
        qd, kd, vd = qkv[:, :C], qkv[:, C:2 * C], qkv[:, 2 * C:]
        kv = jnp.dot(xs_, wkv_r[...], preferred_element_type=_f32) + bkv_r[...]
        ks, vs = kv[:, :C], kv[:, C:]
        ep = jnp.dot(ea_r[...].astype(bf), we_r[...],
                     preferred_element_type=_f32) + be_r[...]
        a1 = (qd * kd) * scale
        a2 = (qd * ks) * scale
        a3 = (qd * ep) * scale
        s1 = jnp.sum(a1, axis=1, keepdims=True)
        s2 = jnp.sum(a2, axis=1, keepdims=True)
        s3 = jnp.sum(a3, axis=1, keepdims=True)
        m = (s1 + s2 + s3) * (1.0 / (3 * C))
        q1 = jnp.sum((a1 - m) ** 2, axis=1, keepdims=True)
        q2 = jnp.sum((a2 - m) ** 2, axis=1, keepdims=True)
        q3 = jnp.sum((a3 - m) ** 2, axis=1, keepdims=True)
        rstd = lax.rsqrt((q1 + q2 + q3) * (1.0 / (3 * C)) + EPS)
        g1 = jax.nn.sigmoid((a1 - m) * rstd * lnas_r[:, :C]
                            + lnab_r[:, :C])
        g2 = jax.nn.sigmoid((a2 - m) * rstd * lnas_r[:, C:2 * C]
                            + lnab_r[:, C:2 * C])
        g3 = jax.nn.sigmoid((a3 - m) * rstd * lnas_r[:, 2 * C:]
                            + lnab_r[:, 2 * C:])
        msg1 = (jnp.dot(vd.astype(bf), mud_r[...], preferred_element_type=_f32)
                + jnp.dot(vs.astype(bf), mus_r[...], preferred_element_type=_f32)
                + jnp.dot(ep.astype(bf), mue_r[...], preferred_element_type=_f32)
                + mub_r[...])
        p1 = (msg1[:, :C] * g1).astype(bf)
        p2 = (msg1[:, C:2 * C] * g2).astype(bf)
        p3 = (msg1[:, 2 * C:] * g3).astype(bf)
        msg2 = (jnp.dot(p1, mlw_r[:C], preferred_element_type=_f32)
                + jnp.dot(p2, mlw_r[C:2 * C], preferred_element_type=_f32)
                + jnp.dot(p3, mlw_r[2 * C:], preferred_element_type=_f32)
                + mlb_r[...])
        m2 = jnp.mean(msg2, axis=1, keepdims=True)
        v2 = jnp.mean((msg2 - m2) ** 2, axis=1, keepdims=True)
        out = (msg2 - m2) * lax.rsqrt(v2 + EPS) * lnms_r[...] + lnmb_r[...]
        out_r[...] = out.reshape(BE // _GCH, _GCH, C)

    return pl.pallas_call(
        body,
        grid=(grid,),
        in_specs=[
            pl.BlockSpec((BE, D), lambda i: (i, 0)),
            pl.BlockSpec((BE, D), lambda i: (i, 0)),
            pl.BlockSpec((BE, EDGE_DIM), lambda i: (i, 0)),
            _full((D, 3 * C)), _full((1, 3 * C)),
            _full((D, 2 * C)), _full((1, 2 * C)),
            _full((EDGE_DIM, C)), _full((1, C)),
            _full((C, 3 * C)), _full((C, 3 * C)), _full((C, 3 * C)),
            _full((1, 3 * C)), _full((1, 3 * C)), _full((1, 3 * C)),
            _full((3 * C, C)), _full((1, C)),
            _full((1, C)), _full((1, C)),
        ],
        out_specs=[
            pl.BlockSpec((BE // _GCH, _GCH, C), lambda i: (i, 0, 0)),
        ],
        out_shape=[jax.ShapeDtypeStruct((E // _GCH, _GCH, C), _f32)],
    )(xd, xs, ea, wqkv, bqkv, wkv, bkv, we, be, mu_d, mu_s, mu_e,
      mu_b, lna_s, lna_b, ml_w, ml_b, lnm_s, lnm_b)[0]


def _node_stats(agg2):
    grid = N // BN_

    def body(a_r, st_r):
        a = a_r[0] + a_r[1]

        @pl.when(pl.program_id(0) == 0)
        def _():
            st_r[...] = jnp.zeros((8, D), _f32)
        st_r[...] += jnp.concatenate(
            [jnp.sum(a, axis=0)[None], jnp.sum(a * a, axis=0)[None],
             jnp.zeros((6, D), _f32)], axis=0)

    return pl.pallas_call(
        body,
        grid=(grid,),
        in_specs=[pl.BlockSpec((2, BN_, D), lambda i: (0, i, 0))],
        out_specs=[pl.BlockSpec((8, D), lambda i: (0, 0))],
        out_shape=[jax.ShapeDtypeStruct((8, D), _f32)],
    )(agg2)[0]


def _finalize(x_cart, agg_c, stats_c, agg_m, x_mat,
              bn_s, bn_b, skip_w, skip_b, bwo, bwr,
              cg1, cg2, cg_b, cl1, cl2, cl_b):
    grid = N // BN_

    def body(xc_r, ac_r, st_r, am_r, xm_r, bns_r, bnb_r, sw_r, sb_r,
             bwo_r, bwr_r, cg1_r, cg2_r, cgb_r, cl1_r, cl2_r, clb_r,
             o1_r, o2_r):
        agg = ac_r[0] + ac_r[1]
        mean = st_r[0:1, :] * (1.0 / N)
        var = st_r[1:2, :] * (1.0 / N) - mean * mean
        xhat = (agg - mean) * lax.rsqrt(var + EPS) * bns_r[...] + bnb_r[...]
        x_c = xc_r[...] + _silu(xhat)

        out = am_r[0] + am_r[1]
        x_r = jnp.dot(xm_r[...], sw_r[...], preferred_element_type=_f32) + sb_r[...]
        blogit = (jnp.dot(out, bwo_r[...], preferred_element_type=_f32)
                  + jnp.dot(x_r, bwr_r[...], preferred_element_type=_f32))
        beta = jax.nn.sigmoid(blogit)
        x_m = beta * x_r + (1.0 - beta) * out

        gate = jax.nn.sigmoid(
            jnp.dot(x_c, cg1_r[...], preferred_element_type=_f32)
            + jnp.dot(x_m, cg2_r[...], preferred_element_type=_f32)
            + cgb_r[...])
        fused = (jnp.dot(x_c, cl1_r[...], preferred_element_type=_f32)
                 + jnp.dot(x_m, cl2_r[...], preferred_element_type=_f32)
                 + clb_r[...])
        x_out = gate * fused + (1.0 - gate) * (x_c + x_m) * 0.5
        o1_r[...] = x_out + x_c
        o2_r[...] = x_out + x_m

    return pl.pallas_call(
        body,
        grid=(grid,),
        in_specs=[
            pl.BlockSpec((BN_, D), lambda i: (i, 0)),
            pl.BlockSpec((2, BN_, D), lambda i: (0, i, 0)),
            _full((8, D)),
            pl.BlockSpec((2, BN_, D), lambda i: (0, i, 0)),
            pl.BlockSpec((BN_, D), lambda i: (i, 0)),
            _full((1, D)), _full((1, D)),
            _full((D, C)), _full((1, C)),
            _full((C, 1)), _full((C, 1)),
            _full((D, D)), _full((D, D)), _full((1, D)),
            _full((D, D)), _full((D, D)), _full((1, D)),
        ],
        out_specs=[
            pl.BlockSpec((BN_, D), lambda i: (i, 0)),
            pl.BlockSpec((BN_, D), lambda i: (i, 0)),
        ],
        out_shape=[
            jax.ShapeDtypeStruct((N, D), _f32),
            jax.ShapeDtypeStruct((N, D), _f32),
        ],
    )(x_cart, agg_c, stats_c, agg_m, x_mat, bn_s, bn_b, skip_w, skip_b,
      bwo, bwr, cg1, cg2, cg_b, cl1, cl2, cl_b)


def kernel(x_cart, edge_index_cart, edge_attr_cart, cart_dist,
           x_mat, edge_index_mat, edge_attr_mat, params):
    p = params

    def _idx3d(v):
        v2 = v.reshape(E // _GCH, _GCH)
        v2 = jnp.pad(v2, ((0, _GCR - E // _GCH), (0, 0)))
        return v2.reshape(_GCR // 8, 8, _GCH)

    src_c3 = _idx3d(edge_index_cart[0])
    dst_c3 = _idx3d(edge_index_cart[1])
    src_m3 = _idx3d(edge_index_mat[0])
    dst_m3 = _idx3d(edge_index_mat[1])

    g1 = p['gate_w1']
    a1 = p['aggr_w1']
    w1d = jnp.concatenate([g1[:D], a1[:D]], axis=1)
    w1s = jnp.concatenate([g1[D:2 * D], a1[D:2 * D]], axis=1)
    w1e = jnp.concatenate([g1[2 * D:], a1[2 * D:]], axis=1)
    b1c = jnp.concatenate([p['gate_b1'], p['aggr_b1']])[None, :]
    b2g = p['gate_b2'][None, :]
    b2a = p['aggr_b2'][None, :]

    bf = jnp.bfloat16
    mu = p['mu_w']
    mu_d, mu_s, mu_e = mu[:C], mu[C:2 * C], mu[2 * C:]
    wqkv = jnp.concatenate([p['wq'], p['wk'], p['wv']], axis=1).astype(bf)
    bqkv = jnp.concatenate([p['bq'], p['bk'], p['bv']])[None, :]
    wkv = jnp.concatenate([p['wk'], p['wv']], axis=1).astype(bf)
    bkv = jnp.concatenate([p['bk'], p['bv']])[None, :]
    bw = p['beta_w']
    bwo = bw[:C] + bw[2 * C:]
    bwr = bw[C:2 * C] - bw[2 * C:]
    cl1, cl2 = p['cl_w'][:D], p['cl_w'][D:]
    cg1, cg2 = p['cg_w'][:D], p['cg_w'][D:]

    dist2 = cart_dist[:, None]

    def _pack(x):
        return lax.bitcast_convert_type(
            x.astype(bf).reshape(N, D // 2, 2), jnp.int32)

    def _unpack(g):
        return lax.bitcast_convert_type(g, bf).reshape(E, D)

    xc_d, xc_s = map(_unpack, _sc_gather_one(_pack(x_cart), dst_c3, src_c3))
    xm_d, xm_s = map(_unpack, _sc_gather_one(_pack(x_mat), dst_m3, src_m3))

    e_gate, msg_pre, e_out, stats_e = _cart_edge1(
        xc_d, xc_s, edge_attr_cart,
        w1d.astype(bf), w1s.astype(bf), w1e.astype(bf), b1c,
        p['gate_w2'].astype(bf), b2g, p['aggr_w2'].astype(bf), b2a)
    msg_c = _cart_edge2(
        e_gate, msg_pre, dist2, stats_e,
        p['bn_edge_s'][None, :], p['bn_edge_b'][None, :])

    msg_m = _mat_edge(
        xm_d, xm_s, edge_attr_mat,
        wqkv, bqkv, wkv, bkv,
        p['we'].astype(bf), p['be'][None, :],
        mu_d.astype(bf), mu_s.astype(bf), mu_e.astype(bf),
        p['mu_b'][None, :],
        p['ln_a_s'][None, :], p['ln_a_b'][None, :],
        p['ml_w'].astype(bf), p['ml_b'][None, :],
        p['ln_m_s'][None, :], p['ln_m_b'][None, :])

    agg_c = _sc_scatter_one(msg_c, dst_c3)
    agg_m = _sc_scatter_one(msg_m, dst_m3)

    stats_n = _node_stats(agg_c)
    o1, o2 = _finalize(
        x_cart, agg_c, stats_n, agg_m, x_mat,
        p['bn_s'][None, :], p['bn_b'][None, :],
        p['skip_w'], p['skip_b'][None, :], bwo, bwr,
        cg1, cg2, p['cg_b'][None, :], cl1, cl2, p['cl_b'][None, :])
    return o1, o2, e_out


# speedup vs baseline: 1.9334x; 1.9334x over previous
import functools
import math

import jax
import jax.numpy as jnp
from jax import lax
from jax.experimental import pallas as pl
from jax.experimental.pallas import tpu as pltpu
from jax.experimental.pallas import tpu_sc as plsc

N = 10000
E = 160000
D = 128
C = 128
EDGE_DIM = 16
RADIUS = 5.0
EPS = 1e-5

BE = 640
BN_ = 400

NC = 2
NS = 16
NW = NC * NS

_f32 = jnp.float32


def _silu(x):
    return x * jax.nn.sigmoid(x)


_GCH = 128
_GW = 5120
_GNC = _GW // _GCH
_GLAST = (E - 31 * _GW) // _GCH
_GCR = E // _GCH + (_GNC - _GLAST)
_GNBUF = 4


def _sc_gather_one(table, idx_d3, idx_s3):
    dt = table.dtype
    W = table.shape[1]
    mesh = plsc.VectorSubcoreMesh(core_axis_name="c", subcore_axis_name="s")

    @functools.partial(
        pl.kernel,
        mesh=mesh,
        out_type=[jax.ShapeDtypeStruct((E // _GCH, _GCH, W), dt)
                  for _ in range(2)],
        scratch_types=[
            pltpu.VMEM((_GNC // 8, 8, _GCH), jnp.int32),
        ] + [pltpu.VMEM((_GCH, W), dt) for _ in range(_GNBUF)] + [
            pltpu.SemaphoreType.DMA for _ in range(2 * _GNBUF)
        ],
    )
    def gather_k(tab_h, id_h, is_h, o_d, o_s, idxb, *bufs_sems):
        rows = bufs_sems[:_GNBUF]
        gsem = bufs_sems[_GNBUF:2 * _GNBUF]
        wsem = bufs_sems[2 * _GNBUF:3 * _GNBUF]
        wid = lax.axis_index("s") * NC + lax.axis_index("c")
        full = wid < NW - 1
        crb = wid * (_GNC // 8)

        for idx_h, out_h in ((id_h, o_d), (is_h, o_s)):
            pltpu.sync_copy(idx_h.at[pl.ds(crb, _GNC // 8)], idxb)

            def wait_g(sl, out_h=out_h):
                pltpu.make_async_copy(tab_h.at[pl.ds(0, _GCH)],
                                      rows[sl], gsem[sl]).wait()

            def wait_w(sl, out_h=out_h):
                pltpu.make_async_copy(rows[sl], out_h.at[0], wsem[sl]).wait()

            def issue(ci, sl, out_h=out_h):
                pltpu.async_copy(tab_h.at[idxb.at[ci // 8, ci % 8]],
                                 rows[sl], gsem[sl])

            def drain(ci, sl, out_h=out_h):
                wait_g(sl)
                pltpu.async_copy(rows[sl], out_h.at[wid * _GNC + ci],
                                 wsem[sl])

            def guarded(ci, f):
                if ci < _GLAST:
                    f()
                else:
                    pl.when(full)(f)

            for ci in range(_GNC + _GNBUF):
                sl = ci % _GNBUF
                di = ci - _GNBUF
                if di >= 0:
                    guarded(di, lambda di=di, sl=sl: drain(di, sl))
                if ci < _GNC:
                    def start(ci=ci, sl=sl):
                        if ci >= _GNBUF:
                            wait_w(sl)
                        issue(ci, sl)
                    guarded(ci, start)
            for sl in range(_GNBUF):
                wait_w(sl)

    o_d, o_s = gather_k(table, idx_d3, idx_s3)
    return o_d, o_s


_SCH = 128
_ZR = 16
_NPT = 624
_NREM = N - _NPT * NS


def _sc_scatter_one(msg, dst2):
    mesh = plsc.VectorSubcoreMesh(core_axis_name="c", subcore_axis_name="s")

    @functools.partial(
        pl.kernel,
        mesh=mesh,
        out_type=[jax.ShapeDtypeStruct((2, N, D), _f32)],
        scratch_types=[
            pltpu.VMEM((_GNC // 8, 8, _SCH), jnp.int32),
            pltpu.VMEM((_SCH, D), _f32),
            pltpu.VMEM((_SCH, D), _f32),
            pltpu.VMEM((_ZR, D), _f32),
            pltpu.VMEM_SHARED((N, D), _f32),
            pltpu.SemaphoreType.DMA,
            pltpu.SemaphoreType.DMA,
        ],
    )
    def scatter_k(m_h, d_h, out_h, idxb, m0, m1, z_v, acc, l0, l1):
        cid = lax.axis_index("c")
        sid = lax.axis_index("s")
        wid = sid * NC + cid
        full = wid < NW - 1
        for r in range(_ZR):
            for cc in range(D // 16):
                z_v[r, pl.ds(cc * 16, 16)] = jnp.zeros((16,), _f32)
        row0 = pl.multiple_of(sid * _NPT, 8)

        def zbody(j, carry):
            pltpu.sync_copy(z_v, acc.at[pl.ds(row0 + j * _ZR, _ZR)])
            return carry
        lax.fori_loop(0, _NPT // _ZR, zbody, 0)

        @pl.when(sid == NS - 1)
        def _():
            pltpu.sync_copy(z_v, acc.at[pl.ds(_NPT * NS, _NREM)])
        plsc.subcore_barrier()

        crb = wid * (_GNC // 8)
        pltpu.sync_copy(d_h.at[pl.ds(crb, _GNC // 8)], idxb)
        mb = (m0, m1)
        ls = (l0, l1)

        def load(ci, sl):
            pltpu.async_copy(m_h.at[wid * _GNC + ci], mb[sl], ls[sl])

        def scat(ci, sl):
            pltpu.make_async_copy(m_h.at[0], mb[sl], ls[sl]).wait()
            pltpu.sync_copy(mb[sl], acc.at[idxb.at[ci // 8, ci % 8]],
                            add=True)

        def guarded(ci, f):
            if ci < _GLAST:
                f()
            else:
                pl.when(full)(f)

        for ci in range(_GNC + 1):
            sl = ci % 2
            if ci < _GNC:
                guarded(ci, lambda ci=ci, sl=sl: load(ci, sl))
            di = ci - 1
            if di >= 0:
                guarded(di, lambda di=di, sl=1 - sl: scat(di, sl))

        plsc.subcore_barrier()

        pltpu.sync_copy(acc.at[pl.ds(row0, _NPT)],
                        out_h.at[cid, pl.ds(row0, _NPT)])

        @pl.when(sid == NS - 1)
        def _():
            pltpu.sync_copy(acc.at[pl.ds(_NPT * NS, _NREM)],
                            out_h.at[cid, pl.ds(_NPT * NS, _NREM)])

    return scatter_k(msg, dst2)[0]


def _full(shape):
    return pl.BlockSpec(shape, lambda i: tuple(0 for _ in shape))


def _cart_edge1(xd, xs, e, w1d, w1s, w1e, b1c, w2g, b2g, w2a, b2a):
    grid = E // BE
    bf = jnp.bfloat16

    def body(xd_r, xs_r, e_r, w1d_r, w1s_r, w1e_r, b1_r, w2g_r, b2g_r,
             w2a_r, b2a_r, eg_r, mp_r, eo_r, st_r):
        e_ = e_r[...]
        xd_ = xd_r[...].reshape(BE, D).astype(bf)
        xs_ = xs_r[...].reshape(BE, D).astype(bf)
        h = (jnp.dot(xd_, w1d_r[...], preferred_element_type=_f32)
             + jnp.dot(xs_, w1s_r[...], preferred_element_type=_f32)
             + jnp.dot(e_.astype(bf), w1e_r[...],
                       preferred_element_type=_f32)
             + b1_r[...])
        eg = jnp.dot(_silu(h[:, :D]).astype(bf), w2g_r[...],
                     preferred_element_type=_f32) + b2g_r[...]
        mp = jnp.dot(_silu(h[:, D:]).astype(bf), w2a_r[...],
                     preferred_element_type=_f32) + b2a_r[...]
        eg_r[...] = eg.astype(bf)
        mp_r[...] = mp.astype(bf)
        eo_r[...] = e_ + eg

        @pl.when(pl.program_id(0) == 0)
        def _():
            st_r[...] = jnp.zeros((8, D), _f32)
        upd = jnp.concatenate(
            [jnp.sum(eg, axis=0)[None], jnp.sum(eg * eg, axis=0)[None],
             jnp.zeros((6, D), _f32)], axis=0)
        st_r[...] += upd

    return pl.pallas_call(
        body,
        grid=(grid,),
        in_specs=[
            pl.BlockSpec((BE // _GCH, _GCH, D), lambda i: (i, 0, 0)),
            pl.BlockSpec((BE // _GCH, _GCH, D), lambda i: (i, 0, 0)),
            pl.BlockSpec((BE, D), lambda i: (i, 0)),
            _full((D, 2 * D)), _full((D, 2 * D)), _full((D, 2 * D)),
            _full((1, 2 * D)),
            _full((D, D)), _full((1, D)), _full((D, D)), _full((1, D)),
        ],
        out_specs=[
            pl.BlockSpec((BE, D), lambda i: (i, 0)),
            pl.BlockSpec((BE, D), lambda i: (i, 0)),
            pl.BlockSpec((BE, D), lambda i: (i, 0)),
            pl.BlockSpec((8, D), lambda i: (0, 0)),
        ],
        out_shape=[
            jax.ShapeDtypeStruct((E, D), bf),
            jax.ShapeDtypeStruct((E, D), bf),
            jax.ShapeDtypeStruct((E, D), _f32),
            jax.ShapeDtypeStruct((8, D), _f32),
        ],
    )(xd, xs, e, w1d, w1s, w1e, b1c, w2g, b2g, w2a, b2a)


def _cart_edge2(e_gate, msg_pre, dist2, stats, bn_s, bn_b):
    grid = E // BE

    def body(eg_r, mp_r, d_r, st_r, s_r, b_r, msg_r):
        eg = eg_r[...].astype(_f32)
        mean = st_r[0:1, :] * (1.0 / E)
        var = st_r[1:2, :] * (1.0 / E) - mean * mean
        rstd = lax.rsqrt(var + EPS)
        xhat = (eg - mean) * rstd * s_r[...] + b_r[...]
        d = d_r[...]
        env = 0.5 * (jnp.cos((jnp.pi / RADIUS) * d) + 1.0)
        env = jnp.where(d <= RADIUS, env, 0.0)
        msg = mp_r[...].astype(_f32) * (jax.nn.sigmoid(xhat) * env)
        msg_r[...] = msg.reshape(BE // _GCH, _GCH, D)

    return pl.pallas_call(
        body,
        grid=(grid,),
        in_specs=[
            pl.BlockSpec((BE, D), lambda i: (i, 0)),
            pl.BlockSpec((BE, D), lambda i: (i, 0)),
            pl.BlockSpec((BE, 1), lambda i: (i, 0)),
            _full((8, D)), _full((1, D)), _full((1, D)),
        ],
        out_specs=[
            pl.BlockSpec((BE // _GCH, _GCH, D), lambda i: (i, 0, 0)),
        ],
        out_shape=[
            jax.ShapeDtypeStruct((E // _GCH, _GCH, D), _f32),
        ],
    )(e_gate, msg_pre, dist2, stats, bn_s, bn_b)[0]


def _mat_edge(xd, xs, ea, wqkv, bqkv, wkv, bkv, we, be,
              mu_d, mu_s, mu_e, mu_b, lna_s, lna_b, ml_w, ml_b,
              lnm_s, lnm_b):
    grid = E // BE
    scale = 1.0 / math.sqrt(3.0 * C)

    def body(xd_r, xs_r, ea_r, wqkv_r, bqkv_r, wkv_r, bkv_r,
             we_r, be_r, mud_r, mus_r, mue_r, mub_r, lnas_r, lnab_r,
             mlw_r, mlb_r, lnms_r, lnmb_r, out_r):
        bf = jnp.bfloat16
        xd_ = xd_r[...].reshape(BE, D).astype(bf)
        xs_ = xs_r[...].reshape(BE, D).astype(bf)
        qkv = jnp.dot(xd_, wqkv_r[...], preferred_element_type=_f32) + bqkv_r[...]
        qd, kd, vd = qkv[:, :C], qkv[:, C:2 * C], qkv[:, 2 * C:]
        kv = jnp.dot(xs_, wkv_r[...], preferred_element_type=_f32) + bkv_r[...]
        ks, vs = kv[:, :C], kv[:, C:]
        ep = jnp.dot(ea_r[...].astype(bf), we_r[...],
                     preferred_element_type=_f32) + be_r[...]
        a1 = (qd * kd) * scale
        a2 = (qd * ks) * scale
        a3 = (qd * ep) * scale
        s1 = jnp.sum(a1, axis=1, keepdims=True)
        s2 = jnp.sum(a2, axis=1, keepdims=True)
        s3 = jnp.sum(a3, axis=1, keepdims=True)
        m = (s1 + s2 + s3) * (1.0 / (3 * C))
        q1 = jnp.sum((a1 - m) ** 2, axis=1, keepdims=True)
        q2 = jnp.sum((a2 - m) ** 2, axis=1, keepdims=True)
        q3 = jnp.sum((a3 - m) ** 2, axis=1, keepdims=True)
        rstd = lax.rsqrt((q1 + q2 + q3) * (1.0 / (3 * C)) + EPS)
        g1 = jax.nn.sigmoid((a1 - m) * rstd * lnas_r[:, :C]
                            + lnab_r[:, :C])
        g2 = jax.nn.sigmoid((a2 - m) * rstd * lnas_r[:, C:2 * C]
                            + lnab_r[:, C:2 * C])
        g3 = jax.nn.sigmoid((a3 - m) * rstd * lnas_r[:, 2 * C:]
                            + lnab_r[:, 2 * C:])
        msg1 = (jnp.dot(vd.astype(bf), mud_r[...], preferred_element_type=_f32)
                + jnp.dot(vs.astype(bf), mus_r[...], preferred_element_type=_f32)
                + jnp.dot(ep.astype(bf), mue_r[...], preferred_element_type=_f32)
                + mub_r[...])
        p1 = (msg1[:, :C] * g1).astype(bf)
        p2 = (msg1[:, C:2 * C] * g2).astype(bf)
        p3 = (msg1[:, 2 * C:] * g3).astype(bf)
        msg2 = (jnp.dot(p1, mlw_r[:C], preferred_element_type=_f32)
                + jnp.dot(p2, mlw_r[C:2 * C], preferred_element_type=_f32)
                + jnp.dot(p3, mlw_r[2 * C:], preferred_element_type=_f32)
                + mlb_r[...])
        m2 = jnp.mean(msg2, axis=1, keepdims=True)
        v2 = jnp.mean((msg2 - m2) ** 2, axis=1, keepdims=True)
        out = (msg2 - m2) * lax.rsqrt(v2 + EPS) * lnms_r[...] + lnmb_r[...]
        out_r[...] = out.reshape(BE // _GCH, _GCH, C)

    return pl.pallas_call(
        body,
        grid=(grid,),
        in_specs=[
            pl.BlockSpec((BE // _GCH, _GCH, D), lambda i: (i, 0, 0)),
            pl.BlockSpec((BE // _GCH, _GCH, D), lambda i: (i, 0, 0)),
            pl.BlockSpec((BE, EDGE_DIM), lambda i: (i, 0)),
            _full((D, 3 * C)), _full((1, 3 * C)),
            _full((D, 2 * C)), _full((1, 2 * C)),
            _full((EDGE_DIM, C)), _full((1, C)),
            _full((C, 3 * C)), _full((C, 3 * C)), _full((C, 3 * C)),
            _full((1, 3 * C)), _full((1, 3 * C)), _full((1, 3 * C)),
            _full((3 * C, C)), _full((1, C)),
            _full((1, C)), _full((1, C)),
        ],
        out_specs=[
            pl.BlockSpec((BE // _GCH, _GCH, C), lambda i: (i, 0, 0)),
        ],
        out_shape=[jax.ShapeDtypeStruct((E // _GCH, _GCH, C), _f32)],
    )(xd, xs, ea, wqkv, bqkv, wkv, bkv, we, be, mu_d, mu_s, mu_e,
      mu_b, lna_s, lna_b, ml_w, ml_b, lnm_s, lnm_b)[0]


def _node_stats(agg2):
    grid = N // BN_

    def body(a_r, st_r):
        a = a_r[0] + a_r[1]

        @pl.when(pl.program_id(0) == 0)
        def _():
            st_r[...] = jnp.zeros((8, D), _f32)
        st_r[...] += jnp.concatenate(
            [jnp.sum(a, axis=0)[None], jnp.sum(a * a, axis=0)[None],
             jnp.zeros((6, D), _f32)], axis=0)

    return pl.pallas_call(
        body,
        grid=(grid,),
        in_specs=[pl.BlockSpec((2, BN_, D), lambda i: (0, i, 0))],
        out_specs=[pl.BlockSpec((8, D), lambda i: (0, 0))],
        out_shape=[jax.ShapeDtypeStruct((8, D), _f32)],
    )(agg2)[0]


def _finalize(x_cart, agg_c, stats_c, agg_m, x_mat,
              bn_s, bn_b, skip_w, skip_b, bwo, bwr,
              cg1, cg2, cg_b, cl1, cl2, cl_b):
    grid = N // BN_

    def body(xc_r, ac_r, st_r, am_r, xm_r, bns_r, bnb_r, sw_r, sb_r,
             bwo_r, bwr_r, cg1_r, cg2_r, cgb_r, cl1_r, cl2_r, clb_r,
             o1_r, o2_r):
        agg = ac_r[0] + ac_r[1]
        mean = st_r[0:1, :] * (1.0 / N)
        var = st_r[1:2, :] * (1.0 / N) - mean * mean
        xhat = (agg - mean) * lax.rsqrt(var + EPS) * bns_r[...] + bnb_r[...]
        x_c = xc_r[...] + _silu(xhat)

        out = am_r[0] + am_r[1]
        x_r = jnp.dot(xm_r[...], sw_r[...], preferred_element_type=_f32) + sb_r[...]
        blogit = (jnp.dot(out, bwo_r[...], preferred_element_type=_f32)
                  + jnp.dot(x_r, bwr_r[...], preferred_element_type=_f32))
        beta = jax.nn.sigmoid(blogit)
        x_m = beta * x_r + (1.0 - beta) * out

        gate = jax.nn.sigmoid(
            jnp.dot(x_c, cg1_r[...], preferred_element_type=_f32)
            + jnp.dot(x_m, cg2_r[...], preferred_element_type=_f32)
            + cgb_r[...])
        fused = (jnp.dot(x_c, cl1_r[...], preferred_element_type=_f32)
                 + jnp.dot(x_m, cl2_r[...], preferred_element_type=_f32)
                 + clb_r[...])
        x_out = gate * fused + (1.0 - gate) * (x_c + x_m) * 0.5
        o1_r[...] = x_out + x_c
        o2_r[...] = x_out + x_m

    return pl.pallas_call(
        body,
        grid=(grid,),
        in_specs=[
            pl.BlockSpec((BN_, D), lambda i: (i, 0)),
            pl.BlockSpec((2, BN_, D), lambda i: (0, i, 0)),
            _full((8, D)),
            pl.BlockSpec((2, BN_, D), lambda i: (0, i, 0)),
            pl.BlockSpec((BN_, D), lambda i: (i, 0)),
            _full((1, D)), _full((1, D)),
            _full((D, C)), _full((1, C)),
            _full((C, 1)), _full((C, 1)),
            _full((D, D)), _full((D, D)), _full((1, D)),
            _full((D, D)), _full((D, D)), _full((1, D)),
        ],
        out_specs=[
            pl.BlockSpec((BN_, D), lambda i: (i, 0)),
            pl.BlockSpec((BN_, D), lambda i: (i, 0)),
        ],
        out_shape=[
            jax.ShapeDtypeStruct((N, D), _f32),
            jax.ShapeDtypeStruct((N, D), _f32),
        ],
    )(x_cart, agg_c, stats_c, agg_m, x_mat, bn_s, bn_b, skip_w, skip_b,
      bwo, bwr, cg1, cg2, cg_b, cl1, cl2, cl_b)


def kernel(x_cart, edge_index_cart, edge_attr_cart, cart_dist,
           x_mat, edge_index_mat, edge_attr_mat, params):
    p = params

    def _idx3d(v):
        v2 = v.reshape(E // _GCH, _GCH)
        v2 = jnp.pad(v2, ((0, _GCR - E // _GCH), (0, 0)))
        return v2.reshape(_GCR // 8, 8, _GCH)

    src_c3 = _idx3d(edge_index_cart[0])
    dst_c3 = _idx3d(edge_index_cart[1])
    src_m3 = _idx3d(edge_index_mat[0])
    dst_m3 = _idx3d(edge_index_mat[1])

    g1 = p['gate_w1']
    a1 = p['aggr_w1']
    w1d = jnp.concatenate([g1[:D], a1[:D]], axis=1)
    w1s = jnp.concatenate([g1[D:2 * D], a1[D:2 * D]], axis=1)
    w1e = jnp.concatenate([g1[2 * D:], a1[2 * D:]], axis=1)
    b1c = jnp.concatenate([p['gate_b1'], p['aggr_b1']])[None, :]
    b2g = p['gate_b2'][None, :]
    b2a = p['aggr_b2'][None, :]

    bf = jnp.bfloat16
    mu = p['mu_w']
    mu_d, mu_s, mu_e = mu[:C], mu[C:2 * C], mu[2 * C:]
    wqkv = jnp.concatenate([p['wq'], p['wk'], p['wv']], axis=1).astype(bf)
    bqkv = jnp.concatenate([p['bq'], p['bk'], p['bv']])[None, :]
    wkv = jnp.concatenate([p['wk'], p['wv']], axis=1).astype(bf)
    bkv = jnp.concatenate([p['bk'], p['bv']])[None, :]
    bw = p['beta_w']
    bwo = bw[:C] + bw[2 * C:]
    bwr = bw[C:2 * C] - bw[2 * C:]
    cl1, cl2 = p['cl_w'][:D], p['cl_w'][D:]
    cg1, cg2 = p['cg_w'][:D], p['cg_w'][D:]

    dist2 = cart_dist[:, None]

    xc_d, xc_s = _sc_gather_one(x_cart, dst_c3, src_c3)
    xm_d, xm_s = _sc_gather_one(x_mat, dst_m3, src_m3)

    e_gate, msg_pre, e_out, stats_e = _cart_edge1(
        xc_d, xc_s, edge_attr_cart,
        w1d.astype(bf), w1s.astype(bf), w1e.astype(bf), b1c,
        p['gate_w2'].astype(bf), b2g, p['aggr_w2'].astype(bf), b2a)
    msg_c = _cart_edge2(
        e_gate, msg_pre, dist2, stats_e,
        p['bn_edge_s'][None, :], p['bn_edge_b'][None, :])

    msg_m = _mat_edge(
        xm_d, xm_s, edge_attr_mat,
        wqkv, bqkv, wkv, bkv,
        p['we'].astype(bf), p['be'][None, :],
        mu_d.astype(bf), mu_s.astype(bf), mu_e.astype(bf),
        p['mu_b'][None, :],
        p['ln_a_s'][None, :], p['ln_a_b'][None, :],
        p['ml_w'].astype(bf), p['ml_b'][None, :],
        p['ln_m_s'][None, :], p['ln_m_b'][None, :])

    agg_c = _sc_scatter_one(msg_c, dst_c3)
    agg_m = _sc_scatter_one(msg_m, dst_m3)

    stats_n = _node_stats(agg_c)
    o1, o2 = _finalize(
        x_cart, agg_c, stats_n, agg_m, x_mat,
        p['bn_s'][None, :], p['bn_b'][None, :],
        p['skip_w'], p['skip_b'][None, :], bwo, bwr,
        cg1, cg2, p['cg_b'][None, :], cl1, cl2, p['cl_b'][None, :])
    return o1, o2, e_out


# speedup vs baseline: 2.0061x; 1.0376x over previous
import functools
import math

import jax
import jax.numpy as jnp
from jax import lax
from jax.experimental import pallas as pl
from jax.experimental.pallas import tpu as pltpu
from jax.experimental.pallas import tpu_sc as plsc

N = 10000
E = 160000
D = 128
C = 128
EDGE_DIM = 16
RADIUS = 5.0
EPS = 1e-5

BE = 640
BN_ = 400

NC = 2
NS = 16
NW = NC * NS

_f32 = jnp.float32


def _silu(x):
    return x * jax.nn.sigmoid(x)


_GCH = 128
_GW = 5120
_GNC = _GW // _GCH
_GLAST = (E - 31 * _GW) // _GCH
_GCR = E // _GCH + (_GNC - _GLAST)
_GNBUF = 4


def _sc_gather_one(table, idx_d3, idx_s3):
    dt = table.dtype
    W = table.shape[1]
    mesh = plsc.VectorSubcoreMesh(core_axis_name="c", subcore_axis_name="s")

    @functools.partial(
        pl.kernel,
        mesh=mesh,
        out_type=[jax.ShapeDtypeStruct((E // _GCH, _GCH, W), dt)
                  for _ in range(2)],
        scratch_types=[
            pltpu.VMEM((_GNC // 8, 8, _GCH), jnp.int32),
        ] + [pltpu.VMEM((_GCH, W), dt) for _ in range(_GNBUF)] + [
            pltpu.SemaphoreType.DMA for _ in range(2 * _GNBUF)
        ],
    )
    def gather_k(tab_h, id_h, is_h, o_d, o_s, idxb, *bufs_sems):
        rows = bufs_sems[:_GNBUF]
        gsem = bufs_sems[_GNBUF:2 * _GNBUF]
        wsem = bufs_sems[2 * _GNBUF:3 * _GNBUF]
        wid = lax.axis_index("s") * NC + lax.axis_index("c")
        full = wid < NW - 1
        crb = wid * (_GNC // 8)

        for idx_h, out_h in ((id_h, o_d), (is_h, o_s)):
            pltpu.sync_copy(idx_h.at[pl.ds(crb, _GNC // 8)], idxb)

            def wait_g(sl, out_h=out_h):
                pltpu.make_async_copy(tab_h.at[pl.ds(0, _GCH)],
                                      rows[sl], gsem[sl]).wait()

            def wait_w(sl, out_h=out_h):
                pltpu.make_async_copy(rows[sl], out_h.at[0], wsem[sl]).wait()

            def issue(ci, sl, out_h=out_h):
                pltpu.async_copy(tab_h.at[idxb.at[ci // 8, ci % 8]],
                                 rows[sl], gsem[sl])

            def drain(ci, sl, out_h=out_h):
                wait_g(sl)
                pltpu.async_copy(rows[sl], out_h.at[wid * _GNC + ci],
                                 wsem[sl])

            def guarded(ci, f):
                if ci < _GLAST:
                    f()
                else:
                    pl.when(full)(f)

            for ci in range(_GNC + _GNBUF):
                sl = ci % _GNBUF
                di = ci - _GNBUF
                if di >= 0:
                    guarded(di, lambda di=di, sl=sl: drain(di, sl))
                if ci < _GNC:
                    def start(ci=ci, sl=sl):
                        if ci >= _GNBUF:
                            wait_w(sl)
                        issue(ci, sl)
                    guarded(ci, start)
            for sl in range(_GNBUF):
                wait_w(sl)

    o_d, o_s = gather_k(table, idx_d3, idx_s3)
    return o_d, o_s


_SCH = 128
_ZR = 16
_NPT = 624
_NREM = N - _NPT * NS


def _sc_scatter_one(msg, dst2):
    mesh = plsc.VectorSubcoreMesh(core_axis_name="c", subcore_axis_name="s")

    @functools.partial(
        pl.kernel,
        mesh=mesh,
        out_type=[jax.ShapeDtypeStruct((2, N, D), _f32)],
        scratch_types=[
            pltpu.VMEM((_GNC // 8, 8, _SCH), jnp.int32),
            pltpu.VMEM((_SCH, D), _f32),
            pltpu.VMEM((_SCH, D), _f32),
            pltpu.VMEM((_ZR, D), _f32),
            pltpu.VMEM_SHARED((N, D), _f32),
            pltpu.SemaphoreType.DMA,
            pltpu.SemaphoreType.DMA,
        ],
    )
    def scatter_k(m_h, d_h, out_h, idxb, m0, m1, z_v, acc, l0, l1):
        cid = lax.axis_index("c")
        sid = lax.axis_index("s")
        wid = sid * NC + cid
        full = wid < NW - 1
        for r in range(_ZR):
            for cc in range(D // 16):
                z_v[r, pl.ds(cc * 16, 16)] = jnp.zeros((16,), _f32)
        row0 = pl.multiple_of(sid * _NPT, 8)

        def zbody(j, carry):
            pltpu.sync_copy(z_v, acc.at[pl.ds(row0 + j * _ZR, _ZR)])
            return carry
        lax.fori_loop(0, _NPT // _ZR, zbody, 0)

        @pl.when(sid == NS - 1)
        def _():
            pltpu.sync_copy(z_v, acc.at[pl.ds(_NPT * NS, _NREM)])
        plsc.subcore_barrier()

        crb = wid * (_GNC // 8)
        pltpu.sync_copy(d_h.at[pl.ds(crb, _GNC // 8)], idxb)
        mb = (m0, m1)
        ls = (l0, l1)

        def load(ci, sl):
            pltpu.async_copy(m_h.at[wid * _GNC + ci], mb[sl], ls[sl])

        def scat(ci, sl):
            pltpu.make_async_copy(m_h.at[0], mb[sl], ls[sl]).wait()
            pltpu.sync_copy(mb[sl], acc.at[idxb.at[ci // 8, ci % 8]],
                            add=True)

        def guarded(ci, f):
            if ci < _GLAST:
                f()
            else:
                pl.when(full)(f)

        for ci in range(_GNC + 1):
            sl = ci % 2
            if ci < _GNC:
                guarded(ci, lambda ci=ci, sl=sl: load(ci, sl))
            di = ci - 1
            if di >= 0:
                guarded(di, lambda di=di, sl=1 - sl: scat(di, sl))

        plsc.subcore_barrier()

        pltpu.sync_copy(acc.at[pl.ds(row0, _NPT)],
                        out_h.at[cid, pl.ds(row0, _NPT)])

        @pl.when(sid == NS - 1)
        def _():
            pltpu.sync_copy(acc.at[pl.ds(_NPT * NS, _NREM)],
                            out_h.at[cid, pl.ds(_NPT * NS, _NREM)])

    return scatter_k(msg, dst2)[0]


def _full(shape):
    return pl.BlockSpec(shape, lambda i: tuple(0 for _ in shape))


def _cart_edge1(xd, xs, e, w1d, w1s, w1e, b1c, w2g, b2g, w2a, b2a):
    grid = E // BE
    bf = jnp.bfloat16

    def body(xd_r, xs_r, e_r, w1d_r, w1s_r, w1e_r, b1_r, w2g_r, b2g_r,
             w2a_r, b2a_r, eg_r, mp_r, eo_r, st_r):
        e_ = e_r[...]
        xd_ = xd_r[...].reshape(BE, D).astype(bf)
        xs_ = xs_r[...].reshape(BE, D).astype(bf)
        h = (jnp.dot(xd_, w1d_r[...], preferred_element_type=_f32)
             + jnp.dot(xs_, w1s_r[...], preferred_element_type=_f32)
             + jnp.dot(e_.astype(bf), w1e_r[...],
                       preferred_element_type=_f32)
             + b1_r[...])
        eg = jnp.dot(_silu(h[:, :D]).astype(bf), w2g_r[...],
                     preferred_element_type=_f32) + b2g_r[...]
        mp = jnp.dot(_silu(h[:, D:]).astype(bf), w2a_r[...],
                     preferred_element_type=_f32) + b2a_r[...]
        eg_r[...] = eg.astype(bf)
        mp_r[...] = mp.astype(bf)
        eo_r[...] = e_ + eg

        @pl.when(pl.program_id(0) == 0)
        def _():
            st_r[...] = jnp.zeros((8, D), _f32)
        upd = jnp.concatenate(
            [jnp.sum(eg, axis=0)[None], jnp.sum(eg * eg, axis=0)[None],
             jnp.zeros((6, D), _f32)], axis=0)
        st_r[...] += upd

    return pl.pallas_call(
        body,
        grid=(grid,),
        in_specs=[
            pl.BlockSpec((BE // _GCH, _GCH, D), lambda i: (i, 0, 0)),
            pl.BlockSpec((BE // _GCH, _GCH, D), lambda i: (i, 0, 0)),
            pl.BlockSpec((BE, D), lambda i: (i, 0)),
            _full((D, 2 * D)), _full((D, 2 * D)), _full((D, 2 * D)),
            _full((1, 2 * D)),
            _full((D, D)), _full((1, D)), _full((D, D)), _full((1, D)),
        ],
        out_specs=[
            pl.BlockSpec((BE, D), lambda i: (i, 0)),
            pl.BlockSpec((BE, D), lambda i: (i, 0)),
            pl.BlockSpec((BE, D), lambda i: (i, 0)),
            pl.BlockSpec((8, D), lambda i: (0, 0)),
        ],
        out_shape=[
            jax.ShapeDtypeStruct((E, D), bf),
            jax.ShapeDtypeStruct((E, D), bf),
            jax.ShapeDtypeStruct((E, D), _f32),
            jax.ShapeDtypeStruct((8, D), _f32),
        ],
    )(xd, xs, e, w1d, w1s, w1e, b1c, w2g, b2g, w2a, b2a)


def _cart_edge2(e_gate, msg_pre, dist2, stats, bn_s, bn_b):
    grid = E // BE

    def body(eg_r, mp_r, d_r, st_r, s_r, b_r, msg_r):
        eg = eg_r[...].astype(_f32)
        mean = st_r[0:1, :] * (1.0 / E)
        var = st_r[1:2, :] * (1.0 / E) - mean * mean
        rstd = lax.rsqrt(var + EPS)
        xhat = (eg - mean) * rstd * s_r[...] + b_r[...]
        d = d_r[...]
        env = 0.5 * (jnp.cos((jnp.pi / RADIUS) * d) + 1.0)
        env = jnp.where(d <= RADIUS, env, 0.0)
        msg = mp_r[...].astype(_f32) * (jax.nn.sigmoid(xhat) * env)
        msg_r[...] = msg.reshape(BE // _GCH, _GCH, D)

    return pl.pallas_call(
        body,
        grid=(grid,),
        in_specs=[
            pl.BlockSpec((BE, D), lambda i: (i, 0)),
            pl.BlockSpec((BE, D), lambda i: (i, 0)),
            pl.BlockSpec((BE, 1), lambda i: (i, 0)),
            _full((8, D)), _full((1, D)), _full((1, D)),
        ],
        out_specs=[
            pl.BlockSpec((BE // _GCH, _GCH, D), lambda i: (i, 0, 0)),
        ],
        out_shape=[
            jax.ShapeDtypeStruct((E // _GCH, _GCH, D), _f32),
        ],
    )(e_gate, msg_pre, dist2, stats, bn_s, bn_b)[0]


def _mat_edge(xd, xs, ea, wqkv, bqkv, wkv, bkv, we, be,
              mu_d, mu_s, mu_e, mu_b, lna_s, lna_b, ml_w, ml_b,
              lnm_s, lnm_b):
    grid = E // BE
    scale = 1.0 / math.sqrt(3.0 * C)

    def body(xd_r, xs_r, ea_r, wqkv_r, bqkv_r, wkv_r, bkv_r,
             we_r, be_r, mud_r, mus_r, mue_r, mub_r, lnas_r, lnab_r,
             mlw_r, mlb_r, lnms_r, lnmb_r, out_r):
        bf = jnp.bfloat16
        xd_ = xd_r[...].reshape(BE, D).astype(bf)
        xs_ = xs_r[...].reshape(BE, D).astype(bf)
        qkv = jnp.dot(xd_, wqkv_r[...], preferred_element_type=_f32) + bqkv_r[...]
        qd, kd, vd = qkv[:, :C], qkv[:, C:2 * C], qkv[:, 2 * C:]
        kv = jnp.dot(xs_, wkv_r[...], preferred_element_type=_f32) + bkv_r[...]
        ks, vs = kv[:, :C], kv[:, C:]
        ep = lax.dot_general(
            ea_r[...].astype(bf), we_r[...],
            dimension_numbers=(((0,), (0,)), ((), ())),
            preferred_element_type=_f32) + be_r[...]
        a1 = (qd * kd) * scale
        a2 = (qd * ks) * scale
        a3 = (qd * ep) * scale
        s1 = jnp.sum(a1, axis=1, keepdims=True)
        s2 = jnp.sum(a2, axis=1, keepdims=True)
        s3 = jnp.sum(a3, axis=1, keepdims=True)
        m = (s1 + s2 + s3) * (1.0 / (3 * C))
        q1 = jnp.sum((a1 - m) ** 2, axis=1, keepdims=True)
        q2 = jnp.sum((a2 - m) ** 2, axis=1, keepdims=True)
        q3 = jnp.sum((a3 - m) ** 2, axis=1, keepdims=True)
        rstd = lax.rsqrt((q1 + q2 + q3) * (1.0 / (3 * C)) + EPS)
        g1 = jax.nn.sigmoid((a1 - m) * rstd * lnas_r[:, :C]
                            + lnab_r[:, :C])
        g2 = jax.nn.sigmoid((a2 - m) * rstd * lnas_r[:, C:2 * C]
                            + lnab_r[:, C:2 * C])
        g3 = jax.nn.sigmoid((a3 - m) * rstd * lnas_r[:, 2 * C:]
                            + lnab_r[:, 2 * C:])
        msg1 = (jnp.dot(vd.astype(bf), mud_r[...], preferred_element_type=_f32)
                + jnp.dot(vs.astype(bf), mus_r[...], preferred_element_type=_f32)
                + jnp.dot(ep.astype(bf), mue_r[...], preferred_element_type=_f32)
                + mub_r[...])
        p1 = (msg1[:, :C] * g1).astype(bf)
        p2 = (msg1[:, C:2 * C] * g2).astype(bf)
        p3 = (msg1[:, 2 * C:] * g3).astype(bf)
        msg2 = (jnp.dot(p1, mlw_r[:C], preferred_element_type=_f32)
                + jnp.dot(p2, mlw_r[C:2 * C], preferred_element_type=_f32)
                + jnp.dot(p3, mlw_r[2 * C:], preferred_element_type=_f32)
                + mlb_r[...])
        m2 = jnp.mean(msg2, axis=1, keepdims=True)
        v2 = jnp.mean((msg2 - m2) ** 2, axis=1, keepdims=True)
        out = (msg2 - m2) * lax.rsqrt(v2 + EPS) * lnms_r[...] + lnmb_r[...]
        out_r[...] = out.reshape(BE // _GCH, _GCH, C)

    return pl.pallas_call(
        body,
        grid=(grid,),
        in_specs=[
            pl.BlockSpec((BE // _GCH, _GCH, D), lambda i: (i, 0, 0)),
            pl.BlockSpec((BE // _GCH, _GCH, D), lambda i: (i, 0, 0)),
            pl.BlockSpec((EDGE_DIM, BE), lambda i: (0, i)),
            _full((D, 3 * C)), _full((1, 3 * C)),
            _full((D, 2 * C)), _full((1, 2 * C)),
            _full((EDGE_DIM, C)), _full((1, C)),
            _full((C, 3 * C)), _full((C, 3 * C)), _full((C, 3 * C)),
            _full((1, 3 * C)), _full((1, 3 * C)), _full((1, 3 * C)),
            _full((3 * C, C)), _full((1, C)),
            _full((1, C)), _full((1, C)),
        ],
        out_specs=[
            pl.BlockSpec((BE // _GCH, _GCH, C), lambda i: (i, 0, 0)),
        ],
        out_shape=[jax.ShapeDtypeStruct((E // _GCH, _GCH, C), _f32)],
    )(xd, xs, ea, wqkv, bqkv, wkv, bkv, we, be, mu_d, mu_s, mu_e,
      mu_b, lna_s, lna_b, ml_w, ml_b, lnm_s, lnm_b)[0]


def _node_stats(agg2):
    grid = N // BN_

    def body(a_r, st_r):
        a = a_r[0] + a_r[1]

        @pl.when(pl.program_id(0) == 0)
        def _():
            st_r[...] = jnp.zeros((8, D), _f32)
        st_r[...] += jnp.concatenate(
            [jnp.sum(a, axis=0)[None], jnp.sum(a * a, axis=0)[None],
             jnp.zeros((6, D), _f32)], axis=0)

    return pl.pallas_call(
        body,
        grid=(grid,),
        in_specs=[pl.BlockSpec((2, BN_, D), lambda i: (0, i, 0))],
        out_specs=[pl.BlockSpec((8, D), lambda i: (0, 0))],
        out_shape=[jax.ShapeDtypeStruct((8, D), _f32)],
    )(agg2)[0]


def _finalize(x_cart, agg_c, stats_c, agg_m, x_mat,
              bn_s, bn_b, skip_w, skip_b, bwo, bwr,
              cg1, cg2, cg_b, cl1, cl2, cl_b):
    grid = N // BN_

    def body(xc_r, ac_r, st_r, am_r, xm_r, bns_r, bnb_r, sw_r, sb_r,
             bwo_r, bwr_r, cg1_r, cg2_r, cgb_r, cl1_r, cl2_r, clb_r,
             o1_r, o2_r):
        agg = ac_r[0] + ac_r[1]
        mean = st_r[0:1, :] * (1.0 / N)
        var = st_r[1:2, :] * (1.0 / N) - mean * mean
        xhat = (agg - mean) * lax.rsqrt(var + EPS) * bns_r[...] + bnb_r[...]
        x_c = xc_r[...] + _silu(xhat)

        out = am_r[0] + am_r[1]
        x_r = jnp.dot(xm_r[...], sw_r[...], preferred_element_type=_f32) + sb_r[...]
        blogit = (jnp.dot(out, bwo_r[...], preferred_element_type=_f32)
                  + jnp.dot(x_r, bwr_r[...], preferred_element_type=_f32))
        beta = jax.nn.sigmoid(blogit)
        x_m = beta * x_r + (1.0 - beta) * out

        gate = jax.nn.sigmoid(
            jnp.dot(x_c, cg1_r[...], preferred_element_type=_f32)
            + jnp.dot(x_m, cg2_r[...], preferred_element_type=_f32)
            + cgb_r[...])
        fused = (jnp.dot(x_c, cl1_r[...], preferred_element_type=_f32)
                 + jnp.dot(x_m, cl2_r[...], preferred_element_type=_f32)
                 + clb_r[...])
        x_out = gate * fused + (1.0 - gate) * (x_c + x_m) * 0.5
        o1_r[...] = x_out + x_c
        o2_r[...] = x_out + x_m

    return pl.pallas_call(
        body,
        grid=(grid,),
        in_specs=[
            pl.BlockSpec((BN_, D), lambda i: (i, 0)),
            pl.BlockSpec((2, BN_, D), lambda i: (0, i, 0)),
            _full((8, D)),
            pl.BlockSpec((2, BN_, D), lambda i: (0, i, 0)),
            pl.BlockSpec((BN_, D), lambda i: (i, 0)),
            _full((1, D)), _full((1, D)),
            _full((D, C)), _full((1, C)),
            _full((C, 1)), _full((C, 1)),
            _full((D, D)), _full((D, D)), _full((1, D)),
            _full((D, D)), _full((D, D)), _full((1, D)),
        ],
        out_specs=[
            pl.BlockSpec((BN_, D), lambda i: (i, 0)),
            pl.BlockSpec((BN_, D), lambda i: (i, 0)),
        ],
        out_shape=[
            jax.ShapeDtypeStruct((N, D), _f32),
            jax.ShapeDtypeStruct((N, D), _f32),
        ],
    )(x_cart, agg_c, stats_c, agg_m, x_mat, bn_s, bn_b, skip_w, skip_b,
      bwo, bwr, cg1, cg2, cg_b, cl1, cl2, cl_b)


def kernel(x_cart, edge_index_cart, edge_attr_cart, cart_dist,
           x_mat, edge_index_mat, edge_attr_mat, params):
    p = params

    def _idx3d(v):
        v2 = v.reshape(E // _GCH, _GCH)
        v2 = jnp.pad(v2, ((0, _GCR - E // _GCH), (0, 0)))
        return v2.reshape(_GCR // 8, 8, _GCH)

    src_c3 = _idx3d(edge_index_cart[0])
    dst_c3 = _idx3d(edge_index_cart[1])
    src_m3 = _idx3d(edge_index_mat[0])
    dst_m3 = _idx3d(edge_index_mat[1])

    g1 = p['gate_w1']
    a1 = p['aggr_w1']
    w1d = jnp.concatenate([g1[:D], a1[:D]], axis=1)
    w1s = jnp.concatenate([g1[D:2 * D], a1[D:2 * D]], axis=1)
    w1e = jnp.concatenate([g1[2 * D:], a1[2 * D:]], axis=1)
    b1c = jnp.concatenate([p['gate_b1'], p['aggr_b1']])[None, :]
    b2g = p['gate_b2'][None, :]
    b2a = p['aggr_b2'][None, :]

    bf = jnp.bfloat16
    mu = p['mu_w']
    mu_d, mu_s, mu_e = mu[:C], mu[C:2 * C], mu[2 * C:]
    wqkv = jnp.concatenate([p['wq'], p['wk'], p['wv']], axis=1).astype(bf)
    bqkv = jnp.concatenate([p['bq'], p['bk'], p['bv']])[None, :]
    wkv = jnp.concatenate([p['wk'], p['wv']], axis=1).astype(bf)
    bkv = jnp.concatenate([p['bk'], p['bv']])[None, :]
    bw = p['beta_w']
    bwo = bw[:C] + bw[2 * C:]
    bwr = bw[C:2 * C] - bw[2 * C:]
    cl1, cl2 = p['cl_w'][:D], p['cl_w'][D:]
    cg1, cg2 = p['cg_w'][:D], p['cg_w'][D:]

    dist2 = cart_dist[:, None]
    eaT = edge_attr_mat.T

    xc_d, xc_s = _sc_gather_one(x_cart, dst_c3, src_c3)
    xm_d, xm_s = _sc_gather_one(x_mat, dst_m3, src_m3)

    e_gate, msg_pre, e_out, stats_e = _cart_edge1(
        xc_d, xc_s, edge_attr_cart,
        w1d.astype(bf), w1s.astype(bf), w1e.astype(bf), b1c,
        p['gate_w2'].astype(bf), b2g, p['aggr_w2'].astype(bf), b2a)
    msg_c = _cart_edge2(
        e_gate, msg_pre, dist2, stats_e,
        p['bn_edge_s'][None, :], p['bn_edge_b'][None, :])

    msg_m = _mat_edge(
        xm_d, xm_s, eaT,
        wqkv, bqkv, wkv, bkv,
        p['we'].astype(bf), p['be'][None, :],
        mu_d.astype(bf), mu_s.astype(bf), mu_e.astype(bf),
        p['mu_b'][None, :],
        p['ln_a_s'][None, :], p['ln_a_b'][None, :],
        p['ml_w'].astype(bf), p['ml_b'][None, :],
        p['ln_m_s'][None, :], p['ln_m_b'][None, :])

    agg_c = _sc_scatter_one(msg_c, dst_c3)
    agg_m = _sc_scatter_one(msg_m, dst_m3)

    stats_n = _node_stats(agg_c)
    o1, o2 = _finalize(
        x_cart, agg_c, stats_n, agg_m, x_mat,
        p['bn_s'][None, :], p['bn_b'][None, :],
        p['skip_w'], p['skip_b'][None, :], bwo, bwr,
        cg1, cg2, p['cg_b'][None, :], cl1, cl2, p['cl_b'][None, :])
    return o1, o2, e_out
